# Initial kernel scaffold; baseline (speedup 1.0000x reference)
import jax
import jax.numpy as jnp
from jax.experimental import pallas as pl


def kernel(x, edge_index, W1, b1, W2, b2):
    raise NotImplementedError("write your pallas kernel here")


# speedup vs baseline: 15.6659x; 15.6659x over previous
import functools

import jax
import jax.numpy as jnp
from jax import lax
from jax.experimental import pallas as pl
from jax.experimental.pallas import tpu as pltpu
from jax.experimental.pallas import tpu_sc as plsc

N = 10000
D = 128
E = 320000

NC = 2
NS = 16
NW = NC * NS
CHUNK = 128
E_PER_W = 10240
E_PAD = NW * E_PER_W
NCHUNK = E_PER_W // CHUNK
N_ACC = 10240
R_PER_TILE = N_ACC // NS

_sc_mesh = plsc.VectorSubcoreMesh(core_axis_name="c", subcore_axis_name="s")


@functools.partial(
    pl.kernel,
    out_type=jax.ShapeDtypeStruct((NC, N_ACC), jnp.float32),
    mesh=_sc_mesh,
    scratch_types=[
        pltpu.VMEM((CHUNK,), jnp.int32),
        pltpu.VMEM((CHUNK,), jnp.float32),
        pltpu.VMEM((R_PER_TILE,), jnp.float32),
        pltpu.VMEM_SHARED((N_ACC,), jnp.float32),
    ],
)
def _sc_degree(dst_hbm, out_hbm, didx, ones_v, zbuf, acc):
    cid = lax.axis_index("c")
    sid = lax.axis_index("s")
    wid = sid * NC + cid
    for i in range(CHUNK // 16):
        ones_v[pl.ds(i * 16, 16)] = jnp.ones((16,), jnp.float32)
    for i in range(R_PER_TILE // 16):
        zbuf[pl.ds(i * 16, 16)] = jnp.zeros((16,), jnp.float32)
    tbase = pl.multiple_of(sid * R_PER_TILE, 8)
    pltpu.sync_copy(zbuf, acc.at[pl.ds(tbase, R_PER_TILE)])
    plsc.subcore_barrier()

    def body(j, carry):
        off = pl.multiple_of(wid * E_PER_W + j * CHUNK, 8)
        pltpu.sync_copy(dst_hbm.at[pl.ds(off, CHUNK)], didx)
        pltpu.sync_copy(ones_v, acc.at[didx], add=True)
        return carry

    lax.fori_loop(0, NCHUNK, body, 0)
    plsc.subcore_barrier()
    pltpu.sync_copy(acc.at[pl.ds(tbase, R_PER_TILE)],
                    out_hbm.at[cid, pl.ds(tbase, R_PER_TILE)])


@functools.partial(
    pl.kernel,
    out_type=jax.ShapeDtypeStruct((NC, N_ACC, D), jnp.float32),
    mesh=_sc_mesh,
    scratch_types=[
        pltpu.VMEM((CHUNK,), jnp.int32),
        pltpu.VMEM((CHUNK,), jnp.int32),
        pltpu.VMEM((CHUNK, D), jnp.float32),
        pltpu.VMEM_SHARED((N_ACC, D), jnp.float32),
        pltpu.SemaphoreType.DMA,
    ],
)
def _sc_propagate(h2_hbm, src_hbm, dst_hbm, zrows_hbm, out_hbm,
                  sidx, didx, rows, acc, sem):
    cid = lax.axis_index("c")
    sid = lax.axis_index("s")
    wid = sid * NC + cid
    rbase = pl.multiple_of(sid * R_PER_TILE, 8)

    @pl.when(cid == 0)
    def _():
        pltpu.sync_copy(h2_hbm.at[pl.ds(rbase, R_PER_TILE)],
                        acc.at[pl.ds(rbase, R_PER_TILE)])

    @pl.when(cid == 1)
    def _():
        pltpu.sync_copy(zrows_hbm.at[pl.ds(rbase, R_PER_TILE)],
                        acc.at[pl.ds(rbase, R_PER_TILE)])

    plsc.subcore_barrier()

    def body(j, carry):
        off = pl.multiple_of(wid * E_PER_W + j * CHUNK, 8)
        pltpu.sync_copy(src_hbm.at[pl.ds(off, CHUNK)], sidx)
        pltpu.sync_copy(dst_hbm.at[pl.ds(off, CHUNK)], didx)
        pltpu.async_copy(h2_hbm.at[sidx], rows, sem).wait()
        pltpu.sync_copy(rows, acc.at[didx], add=True)
        return carry

    lax.fori_loop(0, NCHUNK, body, 0)
    plsc.subcore_barrier()
    pltpu.sync_copy(acc.at[pl.ds(rbase, R_PER_TILE)],
                    out_hbm.at[cid, pl.ds(rbase, R_PER_TILE)])


def _tc_in_body(c0, c1, x, w, h2):
    dis = lax.rsqrt(c0[...] + c1[...] + 1.0)
    h2[...] = jnp.dot(x[...], w[...], preferred_element_type=jnp.float32) * dis


_tc_in = pl.pallas_call(
    _tc_in_body,
    out_shape=jax.ShapeDtypeStruct((N_ACC, D), jnp.float32),
)


def _tc_mid_body(a0, a1, c0, c1, b, w, h2):
    dis = lax.rsqrt(c0[...] + c1[...] + 1.0)
    x2 = jnp.maximum(dis * (a0[...] + a1[...]) + b[...], 0.0)
    h2[...] = jnp.dot(x2, w[...], preferred_element_type=jnp.float32) * dis


_tc_mid = pl.pallas_call(
    _tc_mid_body,
    out_shape=jax.ShapeDtypeStruct((N_ACC, D), jnp.float32),
)


def _tc_out_body(a0, a1, c0, c1, b, out):
    dis = lax.rsqrt(c0[...] + c1[...] + 1.0)
    out[...] = jnp.maximum(dis * (a0[...] + a1[...]) + b[...], 0.0)


_tc_out = pl.pallas_call(
    _tc_out_body,
    out_shape=jax.ShapeDtypeStruct((N_ACC, D), jnp.float32),
)


def kernel(x, edge_index, W1, b1, W2, b2):
    src = edge_index[0].astype(jnp.int32)
    dst = edge_index[1].astype(jnp.int32)
    pad = E_PAD - E
    pad_idx = jnp.arange(pad, dtype=jnp.int32)
    srcp = jnp.concatenate([src, pad_idx % N])
    dstp = jnp.concatenate([dst, N + pad_idx % (N_ACC - N)])

    counts = _sc_degree(dstp)
    c0 = counts[0].reshape(N_ACC, 1)
    c1 = counts[1].reshape(N_ACC, 1)
    xp = jnp.pad(x, ((0, N_ACC - N), (0, 0)))
    zrows = jnp.zeros((N_ACC, D), jnp.float32)
    b1r = b1.reshape(1, D)
    b2r = b2.reshape(1, D)

    h2 = _tc_in(c0, c1, xp, W1)
    a = _sc_propagate(h2, srcp, dstp, zrows)
    h2b = _tc_mid(a[0], a[1], c0, c1, b1r, W2)
    a2 = _sc_propagate(h2b, srcp, dstp, zrows)
    return _tc_out(a2[0], a2[1], c0, c1, b2r)[:N]


# speedup vs baseline: 22.5915x; 1.4421x over previous
import functools

import jax
import jax.numpy as jnp
from jax import lax
from jax.experimental import pallas as pl
from jax.experimental.pallas import tpu as pltpu
from jax.experimental.pallas import tpu_sc as plsc

N = 10000
D = 128
E = 320000

NC = 2
NS = 16
NW = NC * NS
CHUNK = 128
E_PER_W = 10240
E_PAD = NW * E_PER_W
NCHUNK = E_PER_W // CHUNK
N_ACC = 10240
R_PER_TILE = N_ACC // NS
NBUF = 2
NGROUP = NCHUNK // NBUF
GROUP_E = NBUF * CHUNK

_sc_mesh = plsc.VectorSubcoreMesh(core_axis_name="c", subcore_axis_name="s")


@functools.partial(
    pl.kernel,
    out_type=jax.ShapeDtypeStruct((NC, N_ACC), jnp.float32),
    mesh=_sc_mesh,
    scratch_types=[
        pltpu.VMEM((NBUF, CHUNK), jnp.int32),
        pltpu.VMEM((CHUNK,), jnp.float32),
        pltpu.VMEM((R_PER_TILE,), jnp.float32),
        pltpu.VMEM_SHARED((N_ACC,), jnp.float32),
        pltpu.SemaphoreType.DMA((NBUF,)),
    ],
)
def _sc_degree(dst_hbm, out_hbm, didx, ones_v, zbuf, acc, sems):
    cid = lax.axis_index("c")
    sid = lax.axis_index("s")
    wid = sid * NC + cid
    base = wid * E_PER_W
    for i in range(CHUNK // 16):
        ones_v[pl.ds(i * 16, 16)] = jnp.ones((16,), jnp.float32)
    for i in range(R_PER_TILE // 16):
        zbuf[pl.ds(i * 16, 16)] = jnp.zeros((16,), jnp.float32)
    tbase = pl.multiple_of(sid * R_PER_TILE, 8)
    pltpu.sync_copy(zbuf, acc.at[pl.ds(tbase, R_PER_TILE)])
    plsc.subcore_barrier()

    for b in range(NBUF):
        off = pl.multiple_of(base + b * CHUNK, 8)
        pltpu.sync_copy(dst_hbm.at[pl.ds(off, CHUNK)], didx.at[b])

    def group(g, carry):
        for b in range(NBUF):
            pltpu.async_copy(ones_v, acc.at[didx.at[b]], sems.at[b], add=True)

        @pl.when(g < NGROUP - 1)
        def _():
            for b in range(NBUF):
                pltpu.make_async_copy(ones_v, acc.at[didx.at[b]],
                                      sems.at[b]).wait()
                off = pl.multiple_of(base + (g + 1) * GROUP_E + b * CHUNK, 8)
                pltpu.sync_copy(dst_hbm.at[pl.ds(off, CHUNK)], didx.at[b])
        return carry

    lax.fori_loop(0, NGROUP, group, 0)
    for b in range(NBUF):
        pltpu.make_async_copy(ones_v, acc.at[didx.at[b]], sems.at[b]).wait()
    plsc.subcore_barrier()
    pltpu.sync_copy(acc.at[pl.ds(tbase, R_PER_TILE)],
                    out_hbm.at[cid, pl.ds(tbase, R_PER_TILE)])


@functools.partial(
    pl.kernel,
    out_type=jax.ShapeDtypeStruct((NC, N_ACC, D), jnp.float32),
    mesh=_sc_mesh,
    scratch_types=[
        pltpu.VMEM((NBUF, CHUNK), jnp.int32),
        pltpu.VMEM((NBUF, CHUNK), jnp.int32),
        pltpu.VMEM((NBUF, CHUNK, D), jnp.float32),
        pltpu.VMEM_SHARED((N_ACC, D), jnp.float32),
        pltpu.SemaphoreType.DMA((NBUF,)),
        pltpu.SemaphoreType.DMA((NBUF,)),
    ],
)
def _sc_propagate(h2_hbm, src_hbm, dst_hbm, zrows_hbm, out_hbm,
                  sidx, didx, rows, acc, semg, sems):
    cid = lax.axis_index("c")
    sid = lax.axis_index("s")
    wid = sid * NC + cid
    base = wid * E_PER_W
    rbase = pl.multiple_of(sid * R_PER_TILE, 8)

    @pl.when(cid == 0)
    def _():
        pltpu.sync_copy(h2_hbm.at[pl.ds(rbase, R_PER_TILE)],
                        acc.at[pl.ds(rbase, R_PER_TILE)])

    @pl.when(cid == 1)
    def _():
        pltpu.sync_copy(zrows_hbm.at[pl.ds(rbase, R_PER_TILE)],
                        acc.at[pl.ds(rbase, R_PER_TILE)])

    plsc.subcore_barrier()

    for b in range(NBUF):
        off = pl.multiple_of(base + b * CHUNK, 8)
        pltpu.sync_copy(src_hbm.at[pl.ds(off, CHUNK)], sidx.at[b])
        pltpu.sync_copy(dst_hbm.at[pl.ds(off, CHUNK)], didx.at[b])
        pltpu.async_copy(h2_hbm.at[sidx.at[b]], rows.at[b], semg.at[b])

    def group(g, carry):
        for b in range(NBUF):
            pltpu.make_async_copy(h2_hbm.at[sidx.at[b]], rows.at[b],
                                  semg.at[b]).wait()
            pltpu.async_copy(rows.at[b], acc.at[didx.at[b]], sems.at[b],
                             add=True)

        @pl.when(g < NGROUP - 1)
        def _():
            for b in range(NBUF):
                pltpu.make_async_copy(rows.at[b], acc.at[didx.at[b]],
                                      sems.at[b]).wait()
                off = pl.multiple_of(base + (g + 1) * GROUP_E + b * CHUNK, 8)
                pltpu.sync_copy(src_hbm.at[pl.ds(off, CHUNK)], sidx.at[b])
                pltpu.sync_copy(dst_hbm.at[pl.ds(off, CHUNK)], didx.at[b])
                pltpu.async_copy(h2_hbm.at[sidx.at[b]], rows.at[b], semg.at[b])
        return carry

    lax.fori_loop(0, NGROUP, group, 0)
    for b in range(NBUF):
        pltpu.make_async_copy(rows.at[b], acc.at[didx.at[b]], sems.at[b]).wait()
    plsc.subcore_barrier()
    pltpu.sync_copy(acc.at[pl.ds(rbase, R_PER_TILE)],
                    out_hbm.at[cid, pl.ds(rbase, R_PER_TILE)])


def _tc_in_body(c0, c1, x, w, h2):
    dis = lax.rsqrt(c0[...] + c1[...] + 1.0)
    h2[...] = jnp.dot(x[...], w[...], preferred_element_type=jnp.float32) * dis


_tc_in = pl.pallas_call(
    _tc_in_body,
    out_shape=jax.ShapeDtypeStruct((N_ACC, D), jnp.float32),
)


def _tc_mid_body(a0, a1, c0, c1, b, w, h2):
    dis = lax.rsqrt(c0[...] + c1[...] + 1.0)
    x2 = jnp.maximum(dis * (a0[...] + a1[...]) + b[...], 0.0)
    h2[...] = jnp.dot(x2, w[...], preferred_element_type=jnp.float32) * dis


_tc_mid = pl.pallas_call(
    _tc_mid_body,
    out_shape=jax.ShapeDtypeStruct((N_ACC, D), jnp.float32),
)


def _tc_out_body(a0, a1, c0, c1, b, out):
    dis = lax.rsqrt(c0[...] + c1[...] + 1.0)
    out[...] = jnp.maximum(dis * (a0[...] + a1[...]) + b[...], 0.0)


_tc_out = pl.pallas_call(
    _tc_out_body,
    out_shape=jax.ShapeDtypeStruct((N_ACC, D), jnp.float32),
)


def kernel(x, edge_index, W1, b1, W2, b2):
    src = edge_index[0].astype(jnp.int32)
    dst = edge_index[1].astype(jnp.int32)
    pad = E_PAD - E
    pad_idx = jnp.arange(pad, dtype=jnp.int32)
    srcp = jnp.concatenate([src, pad_idx % N])
    dstp = jnp.concatenate([dst, N + pad_idx % (N_ACC - N)])

    counts = _sc_degree(dstp)
    c0 = counts[0].reshape(N_ACC, 1)
    c1 = counts[1].reshape(N_ACC, 1)
    xp = jnp.pad(x, ((0, N_ACC - N), (0, 0)))
    zrows = jnp.zeros((N_ACC, D), jnp.float32)
    b1r = b1.reshape(1, D)
    b2r = b2.reshape(1, D)

    h2 = _tc_in(c0, c1, xp, W1)
    a = _sc_propagate(h2, srcp, dstp, zrows)
    h2b = _tc_mid(a[0], a[1], c0, c1, b1r, W2)
    a2 = _sc_propagate(h2b, srcp, dstp, zrows)
    return _tc_out(a2[0], a2[1], c0, c1, b2r)[:N]


# speedup vs baseline: 23.9192x; 1.0588x over previous
import functools

import jax
import jax.numpy as jnp
from jax import lax
from jax.experimental import pallas as pl
from jax.experimental.pallas import tpu as pltpu
from jax.experimental.pallas import tpu_sc as plsc

N = 10000
D = 128
E = 320000

NC = 2
NS = 16
NW = NC * NS
CHUNK = 128
E_PER_W = 10368
E_PAD = NW * E_PER_W
NCHUNK = E_PER_W // CHUNK
N_ACC = 10112
R_PER_TILE = N_ACC // NS
N_CNT = 10240
C_PER_TILE = N_CNT // NS
NBUF = 3
NGROUP = NCHUNK // NBUF
GROUP_E = NBUF * CHUNK

_sc_mesh = plsc.VectorSubcoreMesh(core_axis_name="c", subcore_axis_name="s")


@functools.partial(
    pl.kernel,
    out_type=jax.ShapeDtypeStruct((NC * N_CNT,), jnp.float32),
    mesh=_sc_mesh,
    scratch_types=[
        pltpu.VMEM((NBUF, CHUNK), jnp.int32),
        pltpu.VMEM((CHUNK,), jnp.float32),
        pltpu.VMEM((C_PER_TILE,), jnp.float32),
        pltpu.VMEM_SHARED((N_CNT,), jnp.float32),
        pltpu.SemaphoreType.DMA((NBUF,)),
    ],
)
def _sc_degree(dst_hbm, out_hbm, didx, ones_v, zbuf, acc, sems):
    cid = lax.axis_index("c")
    sid = lax.axis_index("s")
    wid = sid * NC + cid
    base = wid * E_PER_W
    for i in range(CHUNK // 16):
        ones_v[pl.ds(i * 16, 16)] = jnp.ones((16,), jnp.float32)
    for i in range(C_PER_TILE // 16):
        zbuf[pl.ds(i * 16, 16)] = jnp.zeros((16,), jnp.float32)
    tbase = pl.multiple_of(sid * C_PER_TILE, 8)
    pltpu.sync_copy(zbuf, acc.at[pl.ds(tbase, C_PER_TILE)])
    plsc.subcore_barrier()

    for b in range(NBUF):
        off = pl.multiple_of(base + b * CHUNK, 8)
        pltpu.sync_copy(dst_hbm.at[pl.ds(off, CHUNK)], didx.at[b])

    def group(g, carry):
        for b in range(NBUF):
            pltpu.async_copy(ones_v, acc.at[didx.at[b]], sems.at[b], add=True)

        @pl.when(g < NGROUP - 1)
        def _():
            for b in range(NBUF):
                pltpu.make_async_copy(ones_v, acc.at[didx.at[b]],
                                      sems.at[b]).wait()
                off = pl.multiple_of(base + (g + 1) * GROUP_E + b * CHUNK, 8)
                pltpu.sync_copy(dst_hbm.at[pl.ds(off, CHUNK)], didx.at[b])
        return carry

    lax.fori_loop(0, NGROUP, group, 0)
    for b in range(NBUF):
        pltpu.make_async_copy(ones_v, acc.at[didx.at[b]], sems.at[b]).wait()
    plsc.subcore_barrier()
    obase = pl.multiple_of(cid * N_CNT + tbase, 8)
    pltpu.sync_copy(acc.at[pl.ds(tbase, C_PER_TILE)],
                    out_hbm.at[pl.ds(obase, C_PER_TILE)])


@functools.partial(
    pl.kernel,
    out_type=jax.ShapeDtypeStruct((NC, N_ACC, D), jnp.float32),
    mesh=_sc_mesh,
    scratch_types=[
        pltpu.VMEM((NBUF, CHUNK), jnp.int32),
        pltpu.VMEM((NBUF, CHUNK), jnp.int32),
        pltpu.VMEM((NBUF, CHUNK, D), jnp.float32),
        pltpu.VMEM_SHARED((N_ACC, D), jnp.float32),
        pltpu.SemaphoreType.DMA((NBUF,)),
        pltpu.SemaphoreType.DMA((NBUF,)),
    ],
)
def _sc_propagate(h2_hbm, src_hbm, dst_hbm, zrows_hbm, out_hbm,
                  sidx, didx, rows, acc, semg, sems):
    cid = lax.axis_index("c")
    sid = lax.axis_index("s")
    wid = sid * NC + cid
    base = wid * E_PER_W
    rbase = pl.multiple_of(sid * R_PER_TILE, 8)

    @pl.when(cid == 0)
    def _():
        pltpu.sync_copy(h2_hbm.at[pl.ds(rbase, R_PER_TILE)],
                        acc.at[pl.ds(rbase, R_PER_TILE)])

    @pl.when(cid == 1)
    def _():
        pltpu.sync_copy(zrows_hbm.at[pl.ds(rbase, R_PER_TILE)],
                        acc.at[pl.ds(rbase, R_PER_TILE)])

    plsc.subcore_barrier()

    for b in range(NBUF):
        off = pl.multiple_of(base + b * CHUNK, 8)
        pltpu.sync_copy(src_hbm.at[pl.ds(off, CHUNK)], sidx.at[b])
        pltpu.sync_copy(dst_hbm.at[pl.ds(off, CHUNK)], didx.at[b])
        pltpu.async_copy(h2_hbm.at[sidx.at[b]], rows.at[b], semg.at[b])

    def group(g, carry):
        for b in range(NBUF):
            pltpu.make_async_copy(h2_hbm.at[sidx.at[b]], rows.at[b],
                                  semg.at[b]).wait()
            pltpu.async_copy(rows.at[b], acc.at[didx.at[b]], sems.at[b],
                             add=True)

        @pl.when(g < NGROUP - 1)
        def _():
            for b in range(NBUF):
                pltpu.make_async_copy(rows.at[b], acc.at[didx.at[b]],
                                      sems.at[b]).wait()
                off = pl.multiple_of(base + (g + 1) * GROUP_E + b * CHUNK, 8)
                pltpu.sync_copy(src_hbm.at[pl.ds(off, CHUNK)], sidx.at[b])
                pltpu.sync_copy(dst_hbm.at[pl.ds(off, CHUNK)], didx.at[b])
                pltpu.async_copy(h2_hbm.at[sidx.at[b]], rows.at[b], semg.at[b])
        return carry

    lax.fori_loop(0, NGROUP, group, 0)
    for b in range(NBUF):
        pltpu.make_async_copy(rows.at[b], acc.at[didx.at[b]], sems.at[b]).wait()
    plsc.subcore_barrier()
    pltpu.sync_copy(acc.at[pl.ds(rbase, R_PER_TILE)],
                    out_hbm.at[cid, pl.ds(rbase, R_PER_TILE)])


def _tc_in_body(c0, c1, x, w, h2):
    dis = lax.rsqrt(c0[...] + c1[...] + 1.0)
    h2[...] = jnp.dot(x[...], w[...], preferred_element_type=jnp.float32) * dis


_tc_in = pl.pallas_call(
    _tc_in_body,
    out_shape=jax.ShapeDtypeStruct((N_ACC, D), jnp.float32),
)


def _tc_mid_body(a0, a1, c0, c1, b, w, h2):
    dis = lax.rsqrt(c0[...] + c1[...] + 1.0)
    x2 = jnp.maximum(dis * (a0[...] + a1[...]) + b[...], 0.0)
    h2[...] = jnp.dot(x2, w[...], preferred_element_type=jnp.float32) * dis


_tc_mid = pl.pallas_call(
    _tc_mid_body,
    out_shape=jax.ShapeDtypeStruct((N_ACC, D), jnp.float32),
)


def _tc_out_body(a0, a1, c0, c1, b, out):
    dis = lax.rsqrt(c0[...] + c1[...] + 1.0)
    out[...] = jnp.maximum(dis * (a0[...] + a1[...]) + b[...], 0.0)


_tc_out = pl.pallas_call(
    _tc_out_body,
    out_shape=jax.ShapeDtypeStruct((N_ACC, D), jnp.float32),
)


def kernel(x, edge_index, W1, b1, W2, b2):
    src = edge_index[0].astype(jnp.int32)
    dst = edge_index[1].astype(jnp.int32)
    pad = E_PAD - E
    pad_idx = jnp.arange(pad, dtype=jnp.int32)
    srcp = jnp.concatenate([src, pad_idx % N])
    dstp = jnp.concatenate([dst, N + pad_idx % (N_ACC - N)])

    counts = _sc_degree(dstp)
    c0 = counts[:N_ACC].reshape(N_ACC, 1)
    c1 = counts[N_CNT:N_CNT + N_ACC].reshape(N_ACC, 1)
    xp = jnp.pad(x, ((0, N_ACC - N), (0, 0)))
    zrows = jnp.zeros((N_ACC, D), jnp.float32)
    b1r = b1.reshape(1, D)
    b2r = b2.reshape(1, D)

    h2 = _tc_in(c0, c1, xp, W1)
    a = _sc_propagate(h2, srcp, dstp, zrows)
    h2b = _tc_mid(a[0], a[1], c0, c1, b1r, W2)
    a2 = _sc_propagate(h2b, srcp, dstp, zrows)
    return _tc_out(a2[0], a2[1], c0, c1, b2r)[:N]


# speedup vs baseline: 25.1732x; 1.0524x over previous
import functools

import jax
import jax.numpy as jnp
from jax import lax
from jax.experimental import pallas as pl
from jax.experimental.pallas import tpu as pltpu
from jax.experimental.pallas import tpu_sc as plsc

N = 10000
D = 128
E = 320000

NC = 2
NS = 16
NW = NC * NS
CHUNK = 128
NCHUNK = 80
E_PER_W = NCHUNK * CHUNK
E_PAD = NW * E_PER_W
TOT_CHUNKS = NW * NCHUNK
N_ACC = 10112
R_PER_TILE = N_ACC // NS
N_CNT = 10240
C_PER_TILE = N_CNT // NS
NBUF = 2
NPHASE = 2
CH_PH = NCHUNK // NPHASE
E_PH = CH_PH * CHUNK
NGROUP = CH_PH // NBUF
DEG_WIN = 8

_sc_mesh = plsc.VectorSubcoreMesh(core_axis_name="c", subcore_axis_name="s")


@functools.partial(
    pl.kernel,
    out_type=jax.ShapeDtypeStruct((NC * N_CNT,), jnp.float32),
    mesh=_sc_mesh,
    scratch_types=[
        pltpu.VMEM((NCHUNK, CHUNK), jnp.int32),
        pltpu.VMEM((CHUNK,), jnp.float32),
        pltpu.VMEM((C_PER_TILE,), jnp.float32),
        pltpu.VMEM_SHARED((N_CNT,), jnp.float32),
        pltpu.SemaphoreType.DMA,
    ],
)
def _sc_degree(dst2d_hbm, out_hbm, didx, ones_v, zbuf, acc, sem):
    cid = lax.axis_index("c")
    sid = lax.axis_index("s")
    wid = sid * NC + cid
    for i in range(CHUNK // 16):
        ones_v[pl.ds(i * 16, 16)] = jnp.ones((16,), jnp.float32)
    for i in range(C_PER_TILE // 16):
        zbuf[pl.ds(i * 16, 16)] = jnp.zeros((16,), jnp.float32)
    cbase = pl.multiple_of(wid * NCHUNK, 8)
    pltpu.sync_copy(dst2d_hbm.at[pl.ds(cbase, NCHUNK)], didx)
    tbase = pl.multiple_of(sid * C_PER_TILE, 8)
    pltpu.sync_copy(zbuf, acc.at[pl.ds(tbase, C_PER_TILE)])
    plsc.subcore_barrier()

    def body(j, carry):
        @pl.when(j >= DEG_WIN)
        def _():
            pltpu.make_async_copy(ones_v, acc.at[didx.at[0]], sem).wait()
        pltpu.async_copy(ones_v, acc.at[didx.at[j]], sem, add=True)
        return carry

    lax.fori_loop(0, NCHUNK, body, 0)
    for _ in range(DEG_WIN):
        pltpu.make_async_copy(ones_v, acc.at[didx.at[0]], sem).wait()
    plsc.subcore_barrier()
    obase = pl.multiple_of(cid * N_CNT + tbase, 8)
    pltpu.sync_copy(acc.at[pl.ds(tbase, C_PER_TILE)],
                    out_hbm.at[pl.ds(obase, C_PER_TILE)])


@functools.partial(
    pl.kernel,
    out_type=jax.ShapeDtypeStruct((NC, N_ACC, D), jnp.float32),
    mesh=_sc_mesh,
    scratch_types=[
        pltpu.VMEM((E_PH,), jnp.int32),
        pltpu.VMEM((CH_PH, CHUNK), jnp.int32),
        pltpu.VMEM((NBUF, CHUNK, D), jnp.float32),
        pltpu.VMEM_SHARED((N_ACC, D), jnp.float32),
        pltpu.SemaphoreType.DMA((NBUF,)),
        pltpu.SemaphoreType.DMA((NBUF,)),
    ],
)
def _sc_propagate(h2_hbm, src_hbm, dst2d_hbm, zrows_hbm, out_hbm,
                  sidx, didx, rows, acc, semg, sems):
    cid = lax.axis_index("c")
    sid = lax.axis_index("s")
    wid = sid * NC + cid
    rbase = pl.multiple_of(sid * R_PER_TILE, 8)

    @pl.when(cid == 0)
    def _():
        pltpu.sync_copy(h2_hbm.at[pl.ds(rbase, R_PER_TILE)],
                        acc.at[pl.ds(rbase, R_PER_TILE)])

    @pl.when(cid == 1)
    def _():
        pltpu.sync_copy(zrows_hbm.at[pl.ds(rbase, R_PER_TILE)],
                        acc.at[pl.ds(rbase, R_PER_TILE)])

    plsc.subcore_barrier()

    for p in range(NPHASE):
        ebase = pl.multiple_of(wid * E_PER_W + p * E_PH, 8)
        pltpu.sync_copy(src_hbm.at[pl.ds(ebase, E_PH)], sidx)
        cbase = pl.multiple_of(wid * NCHUNK + p * CH_PH, 8)
        pltpu.sync_copy(dst2d_hbm.at[pl.ds(cbase, CH_PH)], didx)

        for b in range(NBUF):
            pltpu.async_copy(h2_hbm.at[sidx.at[pl.ds(b * CHUNK, CHUNK)]],
                             rows.at[b], semg.at[b])

        def group(g, carry):
            for b in range(NBUF):
                j = g * NBUF + b
                pltpu.make_async_copy(h2_hbm.at[sidx.at[pl.ds(0, CHUNK)]],
                                      rows.at[b], semg.at[b]).wait()
                pltpu.async_copy(rows.at[b], acc.at[didx.at[j]], sems.at[b],
                                 add=True)

            @pl.when(g < NGROUP - 1)
            def _():
                for b in range(NBUF):
                    j = (g + 1) * NBUF + b
                    pltpu.make_async_copy(rows.at[b], acc.at[didx.at[0]],
                                          sems.at[b]).wait()
                    pltpu.async_copy(
                        h2_hbm.at[sidx.at[pl.ds(j * CHUNK, CHUNK)]],
                        rows.at[b], semg.at[b])
            return carry

        lax.fori_loop(0, NGROUP, group, 0)
        for b in range(NBUF):
            pltpu.make_async_copy(rows.at[b], acc.at[didx.at[0]],
                                  sems.at[b]).wait()
    plsc.subcore_barrier()
    pltpu.sync_copy(acc.at[pl.ds(rbase, R_PER_TILE)],
                    out_hbm.at[cid, pl.ds(rbase, R_PER_TILE)])


def _tc_in_body(c0, c1, x, w, h2):
    dis = lax.rsqrt(c0[...] + c1[...] + 1.0)
    h2[...] = jnp.dot(x[...], w[...], preferred_element_type=jnp.float32) * dis


_tc_in = pl.pallas_call(
    _tc_in_body,
    out_shape=jax.ShapeDtypeStruct((N_ACC, D), jnp.float32),
)


def _tc_mid_body(a0, a1, c0, c1, b, w, h2):
    dis = lax.rsqrt(c0[...] + c1[...] + 1.0)
    x2 = jnp.maximum(dis * (a0[...] + a1[...]) + b[...], 0.0)
    h2[...] = jnp.dot(x2, w[...], preferred_element_type=jnp.float32) * dis


_tc_mid = pl.pallas_call(
    _tc_mid_body,
    out_shape=jax.ShapeDtypeStruct((N_ACC, D), jnp.float32),
)


def _tc_out_body(a0, a1, c0, c1, b, out):
    dis = lax.rsqrt(c0[...] + c1[...] + 1.0)
    out[...] = jnp.maximum(dis * (a0[...] + a1[...]) + b[...], 0.0)


_tc_out = pl.pallas_call(
    _tc_out_body,
    out_shape=jax.ShapeDtypeStruct((N_ACC, D), jnp.float32),
)


def kernel(x, edge_index, W1, b1, W2, b2):
    src = edge_index[0].astype(jnp.int32)
    dst = edge_index[1].astype(jnp.int32)
    pad = E_PAD - E
    pad_idx = jnp.arange(pad, dtype=jnp.int32)
    srcp = jnp.concatenate([src, pad_idx % N])
    dstp = jnp.concatenate([dst, N + pad_idx % (N_ACC - N)])
    dst2d = dstp.reshape(TOT_CHUNKS, CHUNK)

    counts = _sc_degree(dst2d)
    c0 = counts[:N_ACC].reshape(N_ACC, 1)
    c1 = counts[N_CNT:N_CNT + N_ACC].reshape(N_ACC, 1)
    xp = jnp.pad(x, ((0, N_ACC - N), (0, 0)))
    zrows = jnp.zeros((N_ACC, D), jnp.float32)
    b1r = b1.reshape(1, D)
    b2r = b2.reshape(1, D)

    h2 = _tc_in(c0, c1, xp, W1)
    a = _sc_propagate(h2, srcp, dst2d, zrows)
    h2b = _tc_mid(a[0], a[1], c0, c1, b1r, W2)
    a2 = _sc_propagate(h2b, srcp, dst2d, zrows)
    return _tc_out(a2[0], a2[1], c0, c1, b2r)[:N]
